# baseline (speedup 1.0000x reference)
import functools

import jax
import jax.numpy as jnp
from jax import lax
from jax.experimental import pallas as pl
from jax.experimental.pallas import tpu as pltpu
from jax.experimental.pallas import tpu_sc as plsc

N = 10000
D = 128
F1 = 32
F2 = 20
NC = 2
NS = 16
NW = NC * NS
CH = 128
E = 320000
NCHUNK = E // CH
CPB = NCHUNK // NW
XTRA = NCHUNK - CPB * NW
SPLIT = NW - XTRA
NG = CPB // 2
RPT = N // NS
DW = 16


def _mesh():
    return plsc.VectorSubcoreMesh(core_axis_name="c", subcore_axis_name="s")


def _chunk0(wid):
    return wid * CPB + jnp.maximum(wid - SPLIT, 0)


def _sc_degrees(srcc, dstc, ones_col, zcol):
    @functools.partial(
        pl.kernel,
        out_type=jax.ShapeDtypeStruct((N, NC, 2, DW), jnp.float32),
        mesh=_mesh(),
        scratch_types=[
            pltpu.VMEM_SHARED((N, DW), jnp.float32),
            pltpu.VMEM_SHARED((N, DW), jnp.float32),
            pltpu.VMEM((CPB + 1, CH), jnp.int32),
            pltpu.VMEM((CPB + 1, CH), jnp.int32),
            pltpu.VMEM((CH, DW), jnp.float32),
            pltpu.SemaphoreType.DMA,
            pltpu.SemaphoreType.DMA,
        ],
        compiler_params=pltpu.CompilerParams(use_tc_tiling_on_sc=False),
    )
    def k(src_h, dst_h, ones_h, z_h, out_h, deg_s, deg_d, sidx, didx, ones_v,
          sem_s, sem_d):
        cid = lax.axis_index("c")
        sid = lax.axis_index("s")
        wid = sid * NC + cid
        c0 = _chunk0(wid)
        pltpu.sync_copy(ones_h, ones_v)
        pltpu.sync_copy(src_h.at[pl.ds(c0, CPB + 1)], sidx)
        pltpu.sync_copy(dst_h.at[pl.ds(c0, CPB + 1)], didx)
        pltpu.sync_copy(z_h.at[pl.ds(sid * RPT, RPT)], deg_s.at[pl.ds(sid * RPT, RPT)])
        pltpu.sync_copy(z_h.at[pl.ds(sid * RPT, RPT)], deg_d.at[pl.ds(sid * RPT, RPT)])
        plsc.subcore_barrier()

        def fire(c, carry):
            pltpu.async_copy(ones_v, deg_s.at[sidx.at[c]], sem_s, add=True)
            pltpu.async_copy(ones_v, deg_d.at[didx.at[c]], sem_d, add=True)
            return carry

        lax.fori_loop(0, CPB, fire, 0)

        @pl.when(wid >= SPLIT)
        def _():
            fire(CPB, 0)

        def drain(c, carry):
            pltpu.make_async_copy(ones_v, deg_s.at[sidx.at[c]], sem_s).wait()
            pltpu.make_async_copy(ones_v, deg_d.at[didx.at[c]], sem_d).wait()
            return carry

        lax.fori_loop(0, CPB, drain, 0)

        @pl.when(wid >= SPLIT)
        def _():
            drain(CPB, 0)

        plsc.subcore_barrier()
        pltpu.sync_copy(deg_s.at[pl.ds(sid * RPT, RPT)],
                        out_h.at[pl.ds(sid * RPT, RPT), cid, 0])
        pltpu.sync_copy(deg_d.at[pl.ds(sid * RPT, RPT)],
                        out_h.at[pl.ds(sid * RPT, RPT), cid, 1])

    return k(srcc, dstc, ones_col, zcol)


def _sc_aggregate(table, srcc, dstc, ewc, zrows):
    @functools.partial(
        pl.kernel,
        out_type=jax.ShapeDtypeStruct((NC, N, F1), jnp.float32),
        mesh=_mesh(),
        scratch_types=[
            pltpu.VMEM_SHARED((N, F1), jnp.float32),
            pltpu.VMEM_SHARED((N, F1), jnp.float32),
            pltpu.VMEM((CPB + 1, CH), jnp.int32),
            pltpu.VMEM((CPB + 1, CH), jnp.int32),
            pltpu.VMEM((CPB + 1, CH), jnp.float32),
            pltpu.VMEM((CH, F1), jnp.float32),
            pltpu.VMEM((CH, F1), jnp.float32),
            pltpu.SemaphoreType.DMA,
            pltpu.SemaphoreType.DMA,
            pltpu.SemaphoreType.DMA,
            pltpu.SemaphoreType.DMA,
        ],
        compiler_params=pltpu.CompilerParams(use_tc_tiling_on_sc=False),
    )
    def k(table_h, src_h, dst_h, ew_h, z_h, out_h, agg_sh, table_sh, sidx, didx, ewv,
          rows0, rows1, gsem0, gsem1, ssem0, ssem1):
        cid = lax.axis_index("c")
        sid = lax.axis_index("s")
        wid = sid * NC + cid
        c0 = _chunk0(wid)
        pltpu.sync_copy(src_h.at[pl.ds(c0, CPB + 1)], sidx)
        pltpu.sync_copy(dst_h.at[pl.ds(c0, CPB + 1)], didx)
        pltpu.sync_copy(ew_h.at[pl.ds(c0, CPB + 1)], ewv)
        pltpu.sync_copy(z_h.at[pl.ds(sid * RPT, RPT)], agg_sh.at[pl.ds(sid * RPT, RPT)])
        pltpu.sync_copy(table_h.at[pl.ds(sid * RPT, RPT)],
                        table_sh.at[pl.ds(sid * RPT, RPT)])
        plsc.subcore_barrier()

        def scale(rows, c):
            for g in range(CH // 16):
                ew16 = ewv[c, pl.ds(g * 16, 16)]
                for l in range(16):
                    e = g * 16 + l
                    w = ew16[l]
                    for h in range(F1 // 16):
                        rows[e, pl.ds(h * 16, 16)] = rows[e, pl.ds(h * 16, 16)] * w

        pltpu.async_copy(table_sh.at[sidx.at[0]], rows0, gsem0)

        def body(g, carry):
            c0_ = 2 * g
            c1 = c0_ + 1

            @pl.when(g > 0)
            def _():
                pltpu.make_async_copy(rows1, agg_sh.at[didx.at[c1 - 2]], ssem1).wait()

            pltpu.async_copy(table_sh.at[sidx.at[c1]], rows1, gsem1)
            pltpu.make_async_copy(table_sh.at[sidx.at[c0_]], rows0, gsem0).wait()
            scale(rows0, c0_)
            pltpu.async_copy(rows0, agg_sh.at[didx.at[c0_]], ssem0, add=True)
            pltpu.make_async_copy(table_sh.at[sidx.at[c1]], rows1, gsem1).wait()
            scale(rows1, c1)
            pltpu.async_copy(rows1, agg_sh.at[didx.at[c1]], ssem1, add=True)

            @pl.when(g < NG - 1)
            def _():
                pltpu.make_async_copy(rows0, agg_sh.at[didx.at[c0_]], ssem0).wait()
                pltpu.async_copy(table_sh.at[sidx.at[c0_ + 2]], rows0, gsem0)

            return carry

        lax.fori_loop(0, NG, body, 0)
        pltpu.make_async_copy(rows0, agg_sh.at[didx.at[CPB - 2]], ssem0).wait()
        pltpu.make_async_copy(rows1, agg_sh.at[didx.at[CPB - 1]], ssem1).wait()

        @pl.when(wid >= SPLIT)
        def _():
            pltpu.async_copy(table_sh.at[sidx.at[CPB]], rows0, gsem0)
            pltpu.make_async_copy(table_sh.at[sidx.at[CPB]], rows0, gsem0).wait()
            scale(rows0, CPB)
            pltpu.async_copy(rows0, agg_sh.at[didx.at[CPB]], ssem0, add=True)
            pltpu.make_async_copy(rows0, agg_sh.at[didx.at[CPB]], ssem0).wait()

        plsc.subcore_barrier()
        pltpu.sync_copy(agg_sh.at[pl.ds(sid * RPT, RPT)],
                        out_h.at[cid, pl.ds(sid * RPT, RPT)])

    return k(table, srcc, dstc, ewc, zrows)


BR = 2000


def _table1_body(x_ref, deg_ref, w_ref, out_ref):
    d = deg_ref[...]
    ns = lax.rsqrt(jnp.maximum(d[:, 0] + d[:, 2], 1.0))
    xb = x_ref[...] * ns[:, None]
    out_ref[...] = jnp.dot(xb, w_ref[...], preferred_element_type=jnp.float32)


def _tc_table1(x, degs2, W1):
    return pl.pallas_call(
        _table1_body,
        grid=(N // BR,),
        in_specs=[
            pl.BlockSpec((BR, D), lambda i: (i, 0)),
            pl.BlockSpec((BR, 4), lambda i: (i, 0)),
            pl.BlockSpec((D, F1), lambda i: (0, 0)),
        ],
        out_specs=pl.BlockSpec((BR, F1), lambda i: (i, 0)),
        out_shape=jax.ShapeDtypeStruct((N, F1), jnp.float32),
    )(x, degs2, W1)


def _mid_body(agg_ref, deg_ref, b1_ref, w2_ref, out_ref):
    a = agg_ref[0] + agg_ref[1]
    d = deg_ref[...]
    nd = lax.rsqrt(jnp.maximum(d[:, 1] + d[:, 3], 1.0))
    ns = lax.rsqrt(jnp.maximum(d[:, 0] + d[:, 2], 1.0))
    t = a * nd[:, None] + b1_ref[...]
    h = jnp.where(t > 0, t, 0.01 * t)
    out_ref[...] = jnp.dot(h * ns[:, None], w2_ref[...],
                           preferred_element_type=jnp.float32)


def _tc_mid(aggs1, degs2, b1r, W2p):
    return pl.pallas_call(
        _mid_body,
        grid=(N // BR,),
        in_specs=[
            pl.BlockSpec((NC, BR, F1), lambda i: (0, i, 0)),
            pl.BlockSpec((BR, 4), lambda i: (i, 0)),
            pl.BlockSpec((1, F1), lambda i: (0, 0)),
            pl.BlockSpec((F1, F1), lambda i: (0, 0)),
        ],
        out_specs=pl.BlockSpec((BR, F1), lambda i: (i, 0)),
        out_shape=jax.ShapeDtypeStruct((N, F1), jnp.float32),
    )(aggs1, degs2, b1r, W2p)


def _final_body(agg_ref, deg_ref, b2_ref, g_ref, bb_ref, f1w_ref, f1b_ref,
                f2w_ref, f2b_ref, out_ref):
    a = agg_ref[0] + agg_ref[1]
    d = deg_ref[...]
    nd = lax.rsqrt(jnp.maximum(d[:, 1] + d[:, 3], 1.0))
    h = a * nd[:, None] + b2_ref[...]
    cmask = lax.broadcasted_iota(jnp.int32, (1, F1), 1) < F2
    m = jnp.where(cmask, 1.0, 0.0)
    mu = jnp.sum(h * m, axis=0, keepdims=True) / N
    var = jnp.sum((h - mu) * (h - mu) * m, axis=0, keepdims=True) / N
    hn = (h - mu) / jnp.sqrt(var + 1e-5) * g_ref[...] + bb_ref[...]
    hl = jnp.where(hn > 0, hn, 0.01 * hn)
    pooled = jnp.sum(hl * m, axis=0, keepdims=True)
    p = pooled[:, :F2]

    def bf(v):
        return v.astype(jnp.bfloat16).astype(jnp.float32)

    o1 = jnp.sum(bf(f1w_ref[...]) * bf(p), axis=1, keepdims=True).T + f1b_ref[...]
    o1 = jnp.where(o1 > 0, o1, 0.01 * o1)
    o2 = jnp.sum(bf(f2w_ref[...]) * bf(o1), axis=1, keepdims=True).T + f2b_ref[...]
    out_ref[...] = o2


def _tc_final(aggs2, degs2, b2p, gp, bp, fc1_w, fc1_br, fc2_w, fc2_br):
    return pl.pallas_call(
        _final_body,
        out_shape=jax.ShapeDtypeStruct((1, 2), jnp.float32),
    )(aggs2, degs2, b2p, gp, bp, fc1_w, fc1_br, fc2_w, fc2_br)


def kernel(x, edge_index, edge_weights, W1, b1, W2, b2, bn_gamma, bn_beta,
           fc1_w, fc1_b, fc2_w, fc2_b):
    srcc = edge_index[0].reshape(NCHUNK, CH)
    dstc = edge_index[1].reshape(NCHUNK, CH)
    ewc = edge_weights.reshape(NCHUNK, CH)
    ones_col = jnp.ones((CH, DW), jnp.float32)
    zcol = jnp.zeros((N, DW), jnp.float32)
    zrows = jnp.zeros((N, F1), jnp.float32)
    W2p = jnp.pad(W2, ((0, 0), (0, F1 - F2)))
    b1r = b1.reshape(1, F1)
    b2p = jnp.pad(b2, (0, F1 - F2)).reshape(1, F1)
    gp = jnp.pad(bn_gamma, (0, F1 - F2)).reshape(1, F1)
    bp = jnp.pad(bn_beta, (0, F1 - F2)).reshape(1, F1)

    degs = _sc_degrees(srcc, dstc, ones_col, zcol)
    degs2 = degs[..., 0].reshape(N, 4)
    table1 = _tc_table1(x, degs2, W1)
    aggs1 = _sc_aggregate(table1, srcc, dstc, ewc, zrows)
    table2 = _tc_mid(aggs1, degs2, b1r, W2p)
    aggs2 = _sc_aggregate(table2, srcc, dstc, ewc, zrows)
    return _tc_final(aggs2, degs2, b2p, gp, bp, fc1_w,
                     fc1_b.reshape(1, 10), fc2_w, fc2_b.reshape(1, 2))


# speedup vs baseline: 1.4031x; 1.4031x over previous
import functools

import jax
import jax.numpy as jnp
from jax import lax
from jax.experimental import pallas as pl
from jax.experimental.pallas import tpu as pltpu
from jax.experimental.pallas import tpu_sc as plsc

N = 10000
NP = 10240
D = 128
F1 = 32
F2 = 20
NC = 2
NS = 16
NW = NC * NS
CH = 128
E = 320000
NCHUNK = E // CH
CPB = NCHUNK // NW
SPLIT = NW - (NCHUNK - CPB * NW)
NG = CPB // 2
RPT = NP // NS
DW = 16


def _mesh():
    return plsc.VectorSubcoreMesh(core_axis_name="c", subcore_axis_name="s")


def _chunk0(wid):
    return wid * CPB + jnp.maximum(wid - SPLIT, 0)


def _sc_degrees(srcp, dstp, ones_col, zcol):
    @functools.partial(
        pl.kernel,
        out_type=jax.ShapeDtypeStruct((NC, 2, NP, DW), jnp.float32),
        mesh=_mesh(),
        scratch_types=[
            pltpu.VMEM_SHARED((NP, DW), jnp.float32),
            pltpu.VMEM_SHARED((NP, DW), jnp.float32),
            pltpu.VMEM((CPB + 1, CH), jnp.int32),
            pltpu.VMEM((CPB + 1, CH), jnp.int32),
            pltpu.VMEM((CH, DW), jnp.float32),
            pltpu.SemaphoreType.DMA,
            pltpu.SemaphoreType.DMA,
        ],
        compiler_params=pltpu.CompilerParams(use_tc_tiling_on_sc=False),
    )
    def k(src_h, dst_h, ones_h, z_h, out_h, deg_s, deg_d, sidx, didx, ones_v,
          sem_s, sem_d):
        cid = lax.axis_index("c")
        sid = lax.axis_index("s")
        wid = sid * NC + cid
        c0 = _chunk0(wid)
        pltpu.sync_copy(ones_h, ones_v)
        pltpu.sync_copy(src_h.at[pl.ds(c0, CPB + 1)], sidx)
        pltpu.sync_copy(dst_h.at[pl.ds(c0, CPB + 1)], didx)
        pltpu.sync_copy(z_h.at[pl.ds(sid * RPT, RPT)], deg_s.at[pl.ds(sid * RPT, RPT)])
        pltpu.sync_copy(z_h.at[pl.ds(sid * RPT, RPT)], deg_d.at[pl.ds(sid * RPT, RPT)])
        plsc.subcore_barrier()

        def fire(c, carry):
            pltpu.async_copy(ones_v, deg_s.at[sidx.at[c]], sem_s, add=True)
            pltpu.async_copy(ones_v, deg_d.at[didx.at[c]], sem_d, add=True)
            return carry

        lax.fori_loop(0, CPB, fire, 0)

        @pl.when(wid >= SPLIT)
        def _():
            fire(CPB, 0)

        def drain(c, carry):
            pltpu.make_async_copy(ones_v, deg_s.at[sidx.at[c]], sem_s).wait()
            pltpu.make_async_copy(ones_v, deg_d.at[didx.at[c]], sem_d).wait()
            return carry

        lax.fori_loop(0, CPB, drain, 0)

        @pl.when(wid >= SPLIT)
        def _():
            drain(CPB, 0)

        plsc.subcore_barrier()
        pltpu.sync_copy(deg_s.at[pl.ds(sid * RPT, RPT)],
                        out_h.at[cid, 0, pl.ds(sid * RPT, RPT)])
        pltpu.sync_copy(deg_d.at[pl.ds(sid * RPT, RPT)],
                        out_h.at[cid, 1, pl.ds(sid * RPT, RPT)])

    return k(srcp, dstp, ones_col, zcol)


def _sc_aggregate(table, srcp, dstp, ewp, zrows):
    @functools.partial(
        pl.kernel,
        out_type=jax.ShapeDtypeStruct((NC, NP, F1), jnp.float32),
        mesh=_mesh(),
        scratch_types=[
            pltpu.VMEM_SHARED((NP, F1), jnp.float32),
            pltpu.VMEM_SHARED((NP, F1), jnp.float32),
            pltpu.VMEM((CPB + 1, CH), jnp.int32),
            pltpu.VMEM((CPB + 1, CH), jnp.int32),
            pltpu.VMEM((CPB + 1, CH), jnp.float32),
            pltpu.VMEM((CH, F1), jnp.float32),
            pltpu.VMEM((CH, F1), jnp.float32),
            pltpu.SemaphoreType.DMA,
            pltpu.SemaphoreType.DMA,
            pltpu.SemaphoreType.DMA,
            pltpu.SemaphoreType.DMA,
        ],
        compiler_params=pltpu.CompilerParams(use_tc_tiling_on_sc=False),
    )
    def k(table_h, src_h, dst_h, ew_h, z_h, out_h, agg_sh, table_sh, sidx, didx, ewv,
          rows0, rows1, gsem0, gsem1, ssem0, ssem1):
        cid = lax.axis_index("c")
        sid = lax.axis_index("s")
        wid = sid * NC + cid
        c0w = _chunk0(wid)
        pltpu.sync_copy(src_h.at[pl.ds(c0w, CPB + 1)], sidx)
        pltpu.sync_copy(dst_h.at[pl.ds(c0w, CPB + 1)], didx)
        pltpu.sync_copy(ew_h.at[pl.ds(c0w, CPB + 1)], ewv)
        pltpu.sync_copy(z_h.at[pl.ds(sid * RPT, RPT)], agg_sh.at[pl.ds(sid * RPT, RPT)])
        pltpu.sync_copy(table_h.at[pl.ds(sid * RPT, RPT)],
                        table_sh.at[pl.ds(sid * RPT, RPT)])
        plsc.subcore_barrier()

        def scale(rows, c):
            for g in range(CH // 16):
                ew16 = ewv[c, pl.ds(g * 16, 16)]
                for l in range(16):
                    e = g * 16 + l
                    w = ew16[l]
                    for h in range(F1 // 16):
                        rows[e, pl.ds(h * 16, 16)] = rows[e, pl.ds(h * 16, 16)] * w

        pltpu.async_copy(table_sh.at[sidx.at[0]], rows0, gsem0)

        def body(g, carry):
            c0 = 2 * g
            c1 = c0 + 1

            @pl.when(g > 0)
            def _():
                pltpu.make_async_copy(rows1, agg_sh.at[didx.at[c1 - 2]], ssem1).wait()

            pltpu.async_copy(table_sh.at[sidx.at[c1]], rows1, gsem1)
            pltpu.make_async_copy(table_sh.at[sidx.at[c0]], rows0, gsem0).wait()
            scale(rows0, c0)
            pltpu.async_copy(rows0, agg_sh.at[didx.at[c0]], ssem0, add=True)
            pltpu.make_async_copy(table_sh.at[sidx.at[c1]], rows1, gsem1).wait()
            scale(rows1, c1)
            pltpu.async_copy(rows1, agg_sh.at[didx.at[c1]], ssem1, add=True)

            @pl.when(g < NG - 1)
            def _():
                pltpu.make_async_copy(rows0, agg_sh.at[didx.at[c0]], ssem0).wait()
                pltpu.async_copy(table_sh.at[sidx.at[c0 + 2]], rows0, gsem0)

            return carry

        lax.fori_loop(0, NG, body, 0)
        pltpu.make_async_copy(rows0, agg_sh.at[didx.at[CPB - 2]], ssem0).wait()
        pltpu.make_async_copy(rows1, agg_sh.at[didx.at[CPB - 1]], ssem1).wait()

        @pl.when(wid >= SPLIT)
        def _():
            pltpu.async_copy(table_sh.at[sidx.at[CPB]], rows0, gsem0)
            pltpu.make_async_copy(table_sh.at[sidx.at[CPB]], rows0, gsem0).wait()
            scale(rows0, CPB)
            pltpu.async_copy(rows0, agg_sh.at[didx.at[CPB]], ssem0, add=True)
            pltpu.make_async_copy(rows0, agg_sh.at[didx.at[CPB]], ssem0).wait()

        plsc.subcore_barrier()
        pltpu.sync_copy(agg_sh.at[pl.ds(sid * RPT, RPT)],
                        out_h.at[cid, pl.ds(sid * RPT, RPT)])

    return k(table, srcp, dstp, ewp, zrows)


BR = 2048


def _table1_body(x_ref, deg_ref, w_ref, out_ref):
    d = deg_ref[...]
    ns = lax.rsqrt(jnp.maximum(d[0, 0] + d[1, 0], 1.0))
    xb = x_ref[...] * ns[:, None]
    out_ref[...] = jnp.dot(xb, w_ref[...], preferred_element_type=jnp.float32)


def _tc_table1(xp, degs2, W1):
    return pl.pallas_call(
        _table1_body,
        grid=(NP // BR,),
        in_specs=[
            pl.BlockSpec((BR, D), lambda i: (i, 0)),
            pl.BlockSpec((2, 2, BR), lambda i: (0, 0, i)),
            pl.BlockSpec((D, F1), lambda i: (0, 0)),
        ],
        out_specs=pl.BlockSpec((BR, F1), lambda i: (i, 0)),
        out_shape=jax.ShapeDtypeStruct((NP, F1), jnp.float32),
    )(xp, degs2, W1)


def _mid_body(agg_ref, deg_ref, b1_ref, w2_ref, out_ref):
    a = agg_ref[0] + agg_ref[1]
    d = deg_ref[...]
    nd = lax.rsqrt(jnp.maximum(d[0, 1] + d[1, 1], 1.0))
    ns = lax.rsqrt(jnp.maximum(d[0, 0] + d[1, 0], 1.0))
    t = a * nd[:, None] + b1_ref[...]
    h = jnp.where(t > 0, t, 0.01 * t)
    out_ref[...] = jnp.dot(h * ns[:, None], w2_ref[...],
                           preferred_element_type=jnp.float32)


def _tc_mid(aggs1, degs2, b1r, W2p):
    return pl.pallas_call(
        _mid_body,
        grid=(NP // BR,),
        in_specs=[
            pl.BlockSpec((NC, BR, F1), lambda i: (0, i, 0)),
            pl.BlockSpec((2, 2, BR), lambda i: (0, 0, i)),
            pl.BlockSpec((1, F1), lambda i: (0, 0)),
            pl.BlockSpec((F1, F1), lambda i: (0, 0)),
        ],
        out_specs=pl.BlockSpec((BR, F1), lambda i: (i, 0)),
        out_shape=jax.ShapeDtypeStruct((NP, F1), jnp.float32),
    )(aggs1, degs2, b1r, W2p)


def _final_body(agg_ref, deg_ref, b2_ref, g_ref, bb_ref, f1w_ref, f1b_ref,
                f2w_ref, f2b_ref, out_ref):
    a = agg_ref[0] + agg_ref[1]
    d = deg_ref[...]
    nd = lax.rsqrt(jnp.maximum(d[0, 1] + d[1, 1], 1.0))
    h = a * nd[:, None] + b2_ref[...]
    rmask = lax.broadcasted_iota(jnp.int32, (NP, 1), 0) < N
    cmask = lax.broadcasted_iota(jnp.int32, (1, F1), 1) < F2
    m = jnp.where(rmask & cmask, 1.0, 0.0)
    mu = jnp.sum(h * m, axis=0, keepdims=True) / N
    var = jnp.sum((h - mu) * (h - mu) * m, axis=0, keepdims=True) / N
    hn = (h - mu) / jnp.sqrt(var + 1e-5) * g_ref[...] + bb_ref[...]
    hl = jnp.where(hn > 0, hn, 0.01 * hn)
    pooled = jnp.sum(hl * m, axis=0, keepdims=True)
    p = pooled[:, :F2]

    def bf(v):
        return v.astype(jnp.bfloat16).astype(jnp.float32)

    o1 = jnp.sum(bf(f1w_ref[...]) * bf(p), axis=1, keepdims=True).T + f1b_ref[...]
    o1 = jnp.where(o1 > 0, o1, 0.01 * o1)
    o2 = jnp.sum(bf(f2w_ref[...]) * bf(o1), axis=1, keepdims=True).T + f2b_ref[...]
    out_ref[...] = o2


def _tc_final(aggs2, degs2, b2p, gp, bp, fc1_w, fc1_br, fc2_w, fc2_br):
    return pl.pallas_call(
        _final_body,
        out_shape=jax.ShapeDtypeStruct((1, 2), jnp.float32),
    )(aggs2, degs2, b2p, gp, bp, fc1_w, fc1_br, fc2_w, fc2_br)


def kernel(x, edge_index, edge_weights, W1, b1, W2, b2, bn_gamma, bn_beta,
           fc1_w, fc1_b, fc2_w, fc2_b):
    srcp = edge_index[0].reshape(NCHUNK, CH)
    dstp = edge_index[1].reshape(NCHUNK, CH)
    ewp = edge_weights.reshape(NCHUNK, CH)
    xp = jnp.pad(x, ((0, NP - N), (0, 0)))
    ones_col = jnp.ones((CH, DW), jnp.float32)
    zcol = jnp.zeros((NP, DW), jnp.float32)
    zrows = jnp.zeros((NP, F1), jnp.float32)
    W2p = jnp.pad(W2, ((0, 0), (0, F1 - F2)))
    b1r = b1.reshape(1, F1)
    b2p = jnp.pad(b2, (0, F1 - F2)).reshape(1, F1)
    gp = jnp.pad(bn_gamma, (0, F1 - F2)).reshape(1, F1)
    bp = jnp.pad(bn_beta, (0, F1 - F2)).reshape(1, F1)

    degs = _sc_degrees(srcp, dstp, ones_col, zcol)
    degs2 = degs[..., 0]
    table1 = _tc_table1(xp, degs2, W1)
    aggs1 = _sc_aggregate(table1, srcp, dstp, ewp, zrows)
    table2 = _tc_mid(aggs1, degs2, b1r, W2p)
    aggs2 = _sc_aggregate(table2, srcp, dstp, ewp, zrows)
    return _tc_final(aggs2, degs2, b2p, gp, bp, fc1_w,
                     fc1_b.reshape(1, 10), fc2_w, fc2_b.reshape(1, 2))
